# Initial kernel scaffold; baseline (speedup 1.0000x reference)
import functools

import jax
import jax.numpy as jnp
from jax import lax
from jax.experimental import pallas as pl
from jax.experimental.pallas import tpu as pltpu
from jax.experimental.pallas import tpu_sc as plsc

EMBED_DIM = 64
NUM_CORES = 2
NUM_SUBCORES = 16
NW = NUM_CORES * NUM_SUBCORES
B_TOTAL = 4096 * 200
B_PER_W = B_TOTAL // NW
CHUNK = 1024
N_CHUNKS = B_PER_W // CHUNK


def _make_kernel():
    mesh = plsc.VectorSubcoreMesh(core_axis_name="c", subcore_axis_name="s")

    @functools.partial(
        pl.kernel,
        mesh=mesh,
        out_type=jax.ShapeDtypeStruct((B_TOTAL, EMBED_DIM), jnp.float32),
        scratch_types=[
            pltpu.VMEM((CHUNK,), jnp.int32),
            pltpu.VMEM((CHUNK, EMBED_DIM), jnp.float32),
            pltpu.SemaphoreType.DMA,
        ],
    )
    def k(t_hbm, p_hbm, out_hbm, idx_v, rows_v, sem):
        wid = lax.axis_index("s") * NUM_CORES + lax.axis_index("c")
        base = wid * B_PER_W

        def body(g, carry):
            off = pl.multiple_of(base + g * CHUNK, 8)
            pltpu.sync_copy(t_hbm.at[pl.ds(off, CHUNK)], idx_v)
            pltpu.async_copy(p_hbm.at[idx_v], rows_v, sem).wait()
            pltpu.sync_copy(rows_v, out_hbm.at[pl.ds(off, CHUNK)])
            return carry

        lax.fori_loop(0, N_CHUNKS, body, 0)

    return k


_gather_kernel = _make_kernel()


def kernel(t, P):
    t_flat = t.reshape(-1).astype(jnp.int32)
    out = _gather_kernel(t_flat, P)
    return out.reshape(t.shape + (P.shape[1],))


# speedup vs baseline: 4.8552x; 4.8552x over previous
import functools

import jax
import jax.numpy as jnp
from jax import lax
from jax.experimental import pallas as pl
from jax.experimental.pallas import tpu as pltpu
from jax.experimental.pallas import tpu_sc as plsc

EMBED_DIM = 64
NUM_CORES = 2
NUM_SUBCORES = 16
NW = NUM_CORES * NUM_SUBCORES
B_TOTAL = 4096 * 200
B_PER_W = B_TOTAL // NW
CHUNK = 1024
N_CHUNKS = B_PER_W // CHUNK


def _make_kernel():
    mesh = plsc.VectorSubcoreMesh(core_axis_name="c", subcore_axis_name="s")

    @functools.partial(
        pl.kernel,
        mesh=mesh,
        out_type=jax.ShapeDtypeStruct((B_TOTAL, EMBED_DIM), jnp.float32),
        scratch_types=[
            pltpu.VMEM((CHUNK,), jnp.int32),
            pltpu.VMEM((CHUNK, EMBED_DIM), jnp.float32),
            pltpu.SemaphoreType.DMA,
        ],
        compiler_params=pltpu.CompilerParams(use_tc_tiling_on_sc=False),
    )
    def k(t_hbm, p_hbm, out_hbm, idx_v, rows_v, sem):
        wid = lax.axis_index("s") * NUM_CORES + lax.axis_index("c")
        base = wid * B_PER_W

        def body(g, carry):
            off = pl.multiple_of(base + g * CHUNK, 8)
            pltpu.sync_copy(t_hbm.at[pl.ds(off, CHUNK)], idx_v)
            pltpu.async_copy(p_hbm.at[idx_v], rows_v, sem).wait()
            pltpu.sync_copy(rows_v, out_hbm.at[pl.ds(off, CHUNK)])
            return carry

        lax.fori_loop(0, N_CHUNKS, body, 0)

    return k


_gather_kernel = _make_kernel()


def kernel(t, P):
    t_flat = t.reshape(-1).astype(jnp.int32)
    out = _gather_kernel(t_flat, P)
    return out.reshape(t.shape + (P.shape[1],))


# speedup vs baseline: 4.9627x; 1.0221x over previous
import functools

import jax
import jax.numpy as jnp
from jax import lax
from jax.experimental import pallas as pl
from jax.experimental.pallas import tpu as pltpu
from jax.experimental.pallas import tpu_sc as plsc

EMBED_DIM = 64
NUM_CORES = 2
NUM_SUBCORES = 16
NW = NUM_CORES * NUM_SUBCORES
B_TOTAL = 4096 * 200
B_PER_W = B_TOTAL // NW
CHUNK = 800
NBUF = 2
N_CHUNKS = B_PER_W // CHUNK
N_OUTER = N_CHUNKS // NBUF


def _make_kernel():
    mesh = plsc.VectorSubcoreMesh(core_axis_name="c", subcore_axis_name="s")

    @functools.partial(
        pl.kernel,
        mesh=mesh,
        out_type=jax.ShapeDtypeStruct((B_TOTAL, EMBED_DIM), jnp.float32),
        scratch_types=[
            pltpu.VMEM((B_PER_W,), jnp.int32),
            pltpu.VMEM((NBUF, CHUNK, EMBED_DIM), jnp.float32),
            pltpu.SemaphoreType.DMA,
            pltpu.SemaphoreType.DMA,
            pltpu.SemaphoreType.DMA,
            pltpu.SemaphoreType.DMA,
        ],
        compiler_params=pltpu.CompilerParams(use_tc_tiling_on_sc=False),
    )
    def k(t_hbm, p_hbm, out_hbm, idx_v, rows_v, sg0, sg1, ss0, ss1):
        wid = lax.axis_index("s") * NUM_CORES + lax.axis_index("c")
        base = wid * B_PER_W
        sem_g = (sg0, sg1)
        sem_s = (ss0, ss1)

        pltpu.sync_copy(t_hbm.at[pl.ds(base, B_PER_W)], idx_v)

        def gather(g, b):
            return pltpu.async_copy(
                p_hbm.at[idx_v.at[pl.ds(g * CHUNK, CHUNK)]],
                rows_v.at[b],
                sem_g[b],
            )

        def store(g, b):
            off = pl.multiple_of(base + g * CHUNK, 8)
            return pltpu.make_async_copy(
                rows_v.at[b],
                out_hbm.at[pl.ds(off, CHUNK)],
                sem_s[b],
            )

        for b in range(NBUF):
            gather(b, b).wait()
            store(b, b).start()

        def body(gg, carry):
            for b in range(NBUF):
                g = gg * NBUF + b
                store(g, b).wait()
                gather(g, b).wait()
                store(g, b).start()
            return carry

        lax.fori_loop(1, N_OUTER, body, 0)

        for b in range(NBUF):
            store(N_CHUNKS - NBUF + b, b).wait()

    return k


_gather_kernel = _make_kernel()


def kernel(t, P):
    t_flat = t.reshape(-1).astype(jnp.int32)
    out = _gather_kernel(t_flat, P)
    return out.reshape(t.shape + (P.shape[1],))
